# baseline (speedup 1.0000x reference)
import functools

import jax
import jax.numpy as jnp
from jax import lax
from jax.experimental import pallas as pl
from jax.experimental.pallas import tpu as pltpu
from jax.experimental.pallas import tpu_sc as plsc

BATCH = 16384
EMBED = 32
LANES = 16
IDX_CHUNK = 128

_info = plsc.get_sparse_core_info()
_NC = _info.num_cores
_NS = _info.num_subcores
_NW = _NC * _NS
_BPW = BATCH // _NW
_NCHUNK = _BPW // IDX_CHUNK


def _sc_body(uid_hbm, iid_hbm, ut_hbm, it_hbm, out_hbm,
             uidx_v, iidx_v, urows_v, irows_v, out_v, sem):
    wid = lax.axis_index("s") * _NC + lax.axis_index("c")
    base = wid * _BPW

    for j in range(_NCHUNK):
        pltpu.sync_copy(uid_hbm.at[pl.ds(base + j * IDX_CHUNK, IDX_CHUNK)],
                        uidx_v.at[j])
        pltpu.sync_copy(iid_hbm.at[pl.ds(base + j * IDX_CHUNK, IDX_CHUNK)],
                        iidx_v.at[j])

    copies = []
    for j in range(_NCHUNK):
        copies.append(pltpu.async_copy(
            ut_hbm.at[uidx_v.at[j]],
            urows_v.at[pl.ds(j * IDX_CHUNK, IDX_CHUNK)], sem))
        copies.append(pltpu.async_copy(
            it_hbm.at[iidx_v.at[j]],
            irows_v.at[pl.ds(j * IDX_CHUNK, IDX_CHUNK)], sem))
    for c in copies:
        c.wait()

    lane = lax.iota(jnp.int32, LANES)

    def group(g, carry):
        acc = jnp.zeros((LANES,), jnp.float32)
        for k in range(LANES):
            b = g * LANES + k
            u0 = urows_v[b, pl.ds(0, LANES)]
            u1 = urows_v[b, pl.ds(LANES, LANES)]
            i0 = irows_v[b, pl.ds(0, LANES)]
            i1 = irows_v[b, pl.ds(LANES, LANES)]
            s = jnp.sum(u0 * i0 + u1 * i1)
            acc = jnp.where(lane == k, s, acc)
        out_v[pl.ds(g * LANES, LANES)] = acc
        return carry

    lax.fori_loop(0, _BPW // LANES, group, 0)

    pltpu.sync_copy(out_v, out_hbm.at[pl.ds(base, _BPW)])


@jax.jit
def _impl(user_ids, item_ids, user_table, item_table):
    mesh = plsc.VectorSubcoreMesh(core_axis_name="c", subcore_axis_name="s")
    f = pl.kernel(
        _sc_body,
        out_type=jax.ShapeDtypeStruct((BATCH,), jnp.float32),
        mesh=mesh,
        compiler_params=pltpu.CompilerParams(
            needs_layout_passes=False, use_tc_tiling_on_sc=False),
        scratch_types=[
            pltpu.VMEM((_NCHUNK, IDX_CHUNK), jnp.int32),
            pltpu.VMEM((_NCHUNK, IDX_CHUNK), jnp.int32),
            pltpu.VMEM((_BPW, EMBED), jnp.float32),
            pltpu.VMEM((_BPW, EMBED), jnp.float32),
            pltpu.VMEM((_BPW,), jnp.float32),
            pltpu.SemaphoreType.DMA,
        ],
    )
    return f(user_ids, item_ids, user_table, item_table)


def kernel(user_ids, item_ids, user_table, item_table):
    return _impl(user_ids.astype(jnp.int32), item_ids.astype(jnp.int32),
                 user_table, item_table)
